# baseline (speedup 1.0000x reference)
import functools

import jax
import jax.numpy as jnp
from jax.experimental import pallas as pl
from jax.experimental.pallas import tpu as pltpu

GD, GH, GW = 41, 320, 320
C_IN = 4
BN_EPS = 1e-3

_VMEM = 56 * 1024 * 1024


def _fold_bn(p):
    Wt, gamma, beta, mean, var = p
    scale = gamma * jax.lax.rsqrt(var + BN_EPS)
    Wf = Wt * scale[:, None, None, None, None]
    b = beta - mean * scale
    return Wf, b


def _ht_for(H):
    if H % 64 == 0 and H >= 192:
        return 64
    if H % 32 == 0 and H >= 96:
        return 32
    return H


def _windows(a, axis, Ht, halo):
    n = (a.shape[axis] - halo) // Ht
    if n == 1:
        return jnp.expand_dims(a, axis), 1
    idx = [slice(None)] * a.ndim
    pieces = []
    for i in range(n):
        s = list(idx)
        s[axis] = slice(i * Ht, i * Ht + Ht + halo)
        pieces.append(a[tuple(s)])
    return jnp.stack(pieces, axis=axis), n


def _subm_kernel(w_ref, b_ref, m_ref, x0_ref, x1_ref, x2_ref, o_ref):
    x_refs = (x0_ref, x1_ref, x2_ref)
    Co, Ht, Ws = o_ref.shape[1], o_ref.shape[3], o_ref.shape[4]
    acc = None
    t = 0
    for kd in range(3):
        xp = x_refs[kd][0, :, 0]
        for kh in range(3):
            for kw in range(3):
                Wk = w_ref[t * Co:(t + 1) * Co, :]
                xs = xp[:, kh:kh + Ht, kw:kw + Ws]
                y = jnp.einsum('oc,chw->ohw', Wk, xs,
                               preferred_element_type=jnp.float32)
                acc = y if acc is None else acc + y
                t += 1
    y = jnp.maximum(acc + b_ref[...], 0.0)
    m = m_ref[0, 0][None, 1:1 + Ht, 1:1 + Ws]
    o_ref[0, :, 0] = (y * m).astype(o_ref.dtype)


def _subm_layer(x_pad, mask_pad, Wf, b, out_dtype=jnp.bfloat16):
    Dp, Ci, Hp, Wp = x_pad.shape
    D, Hs, Ws = Dp - 2, Hp - 2, Wp - 2
    Co = Wf.shape[0]
    Ht = _ht_for(Hs)
    xw, n = _windows(x_pad, 2, Ht, 2)
    mw, _ = _windows(mask_pad, 1, Ht, 2)
    w2 = jnp.transpose(Wf, (2, 3, 4, 0, 1)).reshape(27 * Co, Ci).astype(jnp.bfloat16)
    b3 = b.reshape(Co, 1, 1).astype(jnp.float32)
    in_specs = [
        pl.BlockSpec((27 * Co, Ci), lambda d, t: (0, 0)),
        pl.BlockSpec((Co, 1, 1), lambda d, t: (0, 0, 0)),
        pl.BlockSpec((1, 1, Ht + 2, Wp), lambda d, t: (d + 1, t, 0, 0)),
    ]
    for kd in range(3):
        in_specs.append(
            pl.BlockSpec((1, Ci, 1, Ht + 2, Wp), functools.partial(
                lambda kd, d, t: (d + kd, 0, t, 0, 0), kd)))
    out = pl.pallas_call(
        _subm_kernel,
        grid=(D, n),
        in_specs=in_specs,
        out_specs=pl.BlockSpec((1, Co, 1, Ht, Ws), lambda d, t: (d, 0, t, 0, 0)),
        out_shape=jax.ShapeDtypeStruct((D, Co, n, Ht, Ws), out_dtype),
        compiler_params=pltpu.CompilerParams(
            dimension_semantics=("parallel", "parallel"),
            vmem_limit_bytes=_VMEM),
    )(w2, b3, mw, xw, xw, xw)
    return out.reshape(D, Co, Hs, Ws)


def _down_kernel(taps, nb, w_ref, b_ref, *refs):
    m_refs = refs[:nb]
    x_refs = refs[nb:2 * nb]
    o_ref, mo_ref = refs[2 * nb], refs[2 * nb + 1]
    Co, Ht, Ws = o_ref.shape[1], o_ref.shape[3], o_ref.shape[4]
    acc = None
    mo = None
    for t, (bi, ho, wo) in enumerate(taps):
        xs = x_refs[bi][0, :, 0]
        Wk = w_ref[t * Co:(t + 1) * Co, :]
        y = jnp.einsum('oc,chw->ohw', Wk, xs[:, ho:ho + Ht, wo:wo + Ws],
                       preferred_element_type=jnp.float32)
        acc = y if acc is None else acc + y
        mt = m_refs[bi][0, 0][ho:ho + Ht, wo:wo + Ws]
        mo = mt if mo is None else jnp.maximum(mo, mt)
    y = jnp.maximum(acc + b_ref[...], 0.0)
    o_ref[0, :, 0] = (y * mo[None]).astype(o_ref.dtype)
    mo_ref[0, 0] = mo


def _down_layer(x, mask, Wf, b, stride, pad, ksize, out_dtype=jnp.bfloat16):
    D, Ci, Hs, Ws = x.shape
    Co = Wf.shape[0]
    kd_n, kh_n, kw_n = ksize
    sd, sh, sw = stride
    pd, ph, pw = pad
    Dout = (D + 2 * pd - kd_n) // sd + 1
    Hout = (Hs + 2 * ph - kh_n) // sh + 1 if sh == 2 else Hs
    Wout = (Ws + 2 * pw - kw_n) // sw + 1 if sw == 2 else Ws

    xp = jnp.pad(x, ((pd, pd + 1), (0, 0), (ph, ph), (pw, pw)))
    mp = jnp.pad(mask, ((pd, pd + 1), (ph, ph), (pw, pw)))

    parts, mparts = {}, {}
    for hp_i in range(2 if sh == 2 else 1):
        xh = xp[:, :, hp_i::2, :] if sh == 2 else xp
        mh = mp[:, hp_i::2, :] if sh == 2 else mp
        for wp_i in range(2 if sw == 2 else 1):
            parts[(hp_i, wp_i)] = xh[:, :, :, wp_i::2] if sw == 2 else xh
            mparts[(hp_i, wp_i)] = mh[:, :, wp_i::2] if sw == 2 else mh

    block_key_to_idx = {}
    block_list = []
    taps = []
    w_list = []
    for kd in range(kd_n):
        for kh in range(kh_n):
            for kw in range(kw_n):
                hp_i, ho = (kh % 2, kh // 2) if sh == 2 else (0, kh)
                wp_i, wo = (kw % 2, kw // 2) if sw == 2 else (0, kw)
                key = (hp_i, wp_i, kd)
                if key not in block_key_to_idx:
                    block_key_to_idx[key] = len(block_list)
                    block_list.append(key)
                taps.append((block_key_to_idx[key], ho, wo))
                w_list.append(Wf[:, :, kd, kh, kw])

    nb = len(block_list)
    w2 = jnp.concatenate(w_list, axis=0).astype(jnp.bfloat16)
    b3 = b.reshape(Co, 1, 1).astype(jnp.float32)

    Hc = _ht_for(Hout)
    halo = 1 if sh == 2 else (kh_n - 1)
    Hc_in = Hc if sh == 2 else Hc
    specs = [
        pl.BlockSpec((len(taps) * Co, Ci), lambda d, t: (0, 0)),
        pl.BlockSpec((Co, 1, 1), lambda d, t: (0, 0, 0)),
    ]
    args = []
    n_tiles = None
    for (hp_i, wp_i, kd) in block_list:
        a = mparts[(hp_i, wp_i)]
        aw, n_tiles = _windows(a, 1, Hc, halo)
        specs.append(pl.BlockSpec(
            (1, 1, Hc + halo, aw.shape[3]),
            functools.partial(lambda kd, d, t: (sd * d + kd, t, 0, 0), kd)))
        args.append(aw)
    for (hp_i, wp_i, kd) in block_list:
        a = parts[(hp_i, wp_i)]
        aw, _ = _windows(a, 2, Hc, halo)
        specs.append(pl.BlockSpec(
            (1, Ci, 1, Hc + halo, aw.shape[4]),
            functools.partial(lambda kd, d, t: (sd * d + kd, 0, t, 0, 0), kd)))
        args.append(aw)

    out, mout = pl.pallas_call(
        functools.partial(_down_kernel, taps, nb),
        grid=(Dout, n_tiles),
        in_specs=specs,
        out_specs=[
            pl.BlockSpec((1, Co, 1, Hc, Wout), lambda d, t: (d, 0, t, 0, 0)),
            pl.BlockSpec((1, 1, Hc, Wout), lambda d, t: (d, t, 0, 0)),
        ],
        out_shape=[
            jax.ShapeDtypeStruct((Dout, Co, Hout // Hc, Hc, Wout), out_dtype),
            jax.ShapeDtypeStruct((Dout, Hout // Hc, Hc, Wout), jnp.float32),
        ],
        compiler_params=pltpu.CompilerParams(
            dimension_semantics=("parallel", "parallel"),
            vmem_limit_bytes=_VMEM),
    )(w2, b3, *args)
    return (out.reshape(Dout, Co, Hout, Wout),
            mout.reshape(Dout, Hout, Wout))


def _pad_act(x):
    return jnp.pad(x, ((1, 1), (0, 0), (1, 1), (1, 1)))


def _pad_mask(m):
    return jnp.pad(m, ((1, 1), (1, 1), (1, 1)))


def kernel(voxel_features, voxel_lin_idx, params):
    D, Hs, Ws = GD, GH, GW
    dense = jnp.zeros((C_IN, D * Hs * Ws), jnp.float32).at[:, voxel_lin_idx].set(
        voxel_features.T)
    x = dense.reshape(C_IN, D, Hs, Ws).transpose(1, 0, 2, 3).astype(jnp.bfloat16)
    mask = jnp.zeros((D * Hs * Ws,), jnp.float32).at[voxel_lin_idx].set(
        1.0).reshape(D, Hs, Ws)

    fold = [_fold_bn(p) for p in params]

    xp = _pad_act(x)
    mp = _pad_mask(mask)
    x = _subm_layer(xp, mp, *fold[0])
    x = _subm_layer(_pad_act(x), mp, *fold[1])
    x, mask = _down_layer(x, mask, *fold[2], (2, 2, 2), (1, 1, 1), (3, 3, 3))
    mp = _pad_mask(mask)
    x = _subm_layer(_pad_act(x), mp, *fold[3])
    x = _subm_layer(_pad_act(x), mp, *fold[4])
    x, mask = _down_layer(x, mask, *fold[5], (2, 2, 2), (1, 1, 1), (3, 3, 3))
    return (x, mask)
    x = _subm_layer(_pad_act(x), mp, *fold[1])
    x, mask = _down_layer(x, mask, *fold[2], (2, 2, 2), (1, 1, 1), (3, 3, 3))
    mp = _pad_mask(mask)
    x = _subm_layer(_pad_act(x), mp, *fold[3])
    x = _subm_layer(_pad_act(x), mp, *fold[4])
    x, mask = _down_layer(x, mask, *fold[5], (2, 2, 2), (1, 1, 1), (3, 3, 3))
    mp = _pad_mask(mask)
    x = _subm_layer(_pad_act(x), mp, *fold[6])
    x = _subm_layer(_pad_act(x), mp, *fold[7])
    x, mask = _down_layer(x, mask, *fold[8], (2, 2, 2), (0, 1, 1), (3, 3, 3))
    mp = _pad_mask(mask)
    x = _subm_layer(_pad_act(x), mp, *fold[9])
    x = _subm_layer(_pad_act(x), mp, *fold[10])
    x, mask = _down_layer(x, mask, *fold[11], (2, 1, 1), (0, 0, 0), (3, 1, 1),
                          out_dtype=jnp.float32)
    return x.transpose(1, 0, 2, 3)[None]


# speedup vs baseline: 1.4166x; 1.4166x over previous
import functools

import jax
import jax.numpy as jnp
from jax.experimental import pallas as pl
from jax.experimental.pallas import tpu as pltpu

GD, GH, GW = 41, 320, 320
C_IN = 4
BN_EPS = 1e-3

_VMEM = 56 * 1024 * 1024


def _fold_bn(p):
    Wt, gamma, beta, mean, var = p
    scale = gamma * jax.lax.rsqrt(var + BN_EPS)
    Wf = Wt * scale[:, None, None, None, None]
    b = beta - mean * scale
    return Wf, b


def _geom(H, W):
    Wp, Hp = W + 2, H + 2
    g = Wp + 1
    Lg = Hp * Wp
    Lc = ((g + Lg + g + 127) // 128) * 128
    return Wp, Hp, g, Lg, Lc


def _subm_kernel(Co, Wp, g, Lg, Dp, w_ref, b_ref, m_ref, x0, x1, x2, o_ref):
    d = pl.program_id(0)

    @pl.when((d >= 1) & (d <= Dp - 2))
    def _compute():
        acc = None
        t = 0
        for xr in (x0, x1, x2):
            for kh in range(3):
                for kw in range(3):
                    base = kh * Wp + kw
                    Wk = w_ref[t * Co:(t + 1) * Co, :]
                    xs = xr[0, :, base:base + Lg]
                    y = jnp.dot(Wk, xs, preferred_element_type=jnp.float32)
                    acc = y if acc is None else acc + y
                    t += 1
        y = jnp.maximum(acc + b_ref[...], 0.0) * m_ref[0][:, g:g + Lg]
        o_ref[0, :, :g] = jnp.zeros((Co, g), o_ref.dtype)
        o_ref[0, :, g:g + Lg] = y.astype(o_ref.dtype)
        o_ref[0, :, g + Lg:] = jnp.zeros((Co, o_ref.shape[2] - g - Lg), o_ref.dtype)

    @pl.when((d < 1) | (d > Dp - 2))
    def _zero():
        o_ref[...] = jnp.zeros(o_ref.shape, o_ref.dtype)


def _subm_layer(x, m, Wf, b, H, W, out_dtype=jnp.bfloat16):
    Wp, Hp, g, Lg, Lc = _geom(H, W)
    Dp, Ci = x.shape[0], x.shape[1]
    Co = Wf.shape[0]
    w2 = jnp.transpose(Wf, (2, 3, 4, 0, 1)).reshape(27 * Co, Ci).astype(jnp.bfloat16)
    b3 = b.reshape(Co, 1).astype(jnp.float32)
    in_specs = [
        pl.BlockSpec((27 * Co, Ci), lambda d: (0, 0)),
        pl.BlockSpec((Co, 1), lambda d: (0, 0)),
        pl.BlockSpec((1, 1, Lc), lambda d: (d, 0, 0)),
    ]
    for kd in range(3):
        in_specs.append(pl.BlockSpec(
            (1, Ci, Lc),
            functools.partial(
                lambda kd, d: (jnp.clip(d + kd - 1, 0, Dp - 1), 0, 0), kd)))
    return pl.pallas_call(
        functools.partial(_subm_kernel, Co, Wp, g, Lg, Dp),
        grid=(Dp,),
        in_specs=in_specs,
        out_specs=pl.BlockSpec((1, Co, Lc), lambda d: (d, 0, 0)),
        out_shape=jax.ShapeDtypeStruct((Dp, Co, Lc), out_dtype),
        compiler_params=pltpu.CompilerParams(
            dimension_semantics=("parallel",),
            vmem_limit_bytes=_VMEM),
    )(w2, b3, m, x, x, x)


def _down_kernel(Co, taps, nb, g2, Lg2, Doutp, w_ref, b_ref, i_ref, *refs):
    m_refs = refs[:nb]
    x_refs = refs[nb:2 * nb]
    o_ref, mo_ref = refs[2 * nb], refs[2 * nb + 1]
    d = pl.program_id(0)

    @pl.when((d >= 1) & (d <= Doutp - 2))
    def _compute():
        acc = None
        mo = None
        for t, (bi, base) in enumerate(taps):
            Wk = w_ref[t * Co:(t + 1) * Co, :]
            xs = x_refs[bi][0, :, base:base + Lg2]
            y = jnp.dot(Wk, xs, preferred_element_type=jnp.float32)
            acc = y if acc is None else acc + y
            mt = m_refs[bi][0][:, base:base + Lg2]
            mo = mt if mo is None else jnp.maximum(mo, mt)
        mo = mo * i_ref[...]
        y = jnp.maximum(acc + b_ref[...], 0.0) * mo
        Lc2 = o_ref.shape[2]
        o_ref[0, :, :g2] = jnp.zeros((Co, g2), o_ref.dtype)
        o_ref[0, :, g2:g2 + Lg2] = y.astype(o_ref.dtype)
        o_ref[0, :, g2 + Lg2:] = jnp.zeros((Co, Lc2 - g2 - Lg2), o_ref.dtype)
        mo_ref[0, :, :g2] = jnp.zeros((1, g2), mo_ref.dtype)
        mo_ref[0, :, g2:g2 + Lg2] = mo
        mo_ref[0, :, g2 + Lg2:] = jnp.zeros((1, mo_ref.shape[2] - g2 - Lg2),
                                            mo_ref.dtype)

    @pl.when((d < 1) | (d > Doutp - 2))
    def _zero():
        o_ref[...] = jnp.zeros(o_ref.shape, o_ref.dtype)
        mo_ref[...] = jnp.zeros(mo_ref.shape, mo_ref.dtype)


def _down_layer(x, m, Wf, b, stride, pad, ksize, H, W,
                out_dtype=jnp.bfloat16):
    Wp, Hp, g, Lg, Lc = _geom(H, W)
    Dp, Ci = x.shape[0], x.shape[1]
    D = Dp - 2
    Co = Wf.shape[0]
    kd_n, kh_n, kw_n = ksize
    sd, sh, sw = stride
    pd, ph, pw = pad
    Dout = (D + 2 * pd - kd_n) // sd + 1
    Hout = (H + 2 * ph - kh_n) // sh + 1 if sh == 2 else H
    Wout = (W + 2 * pw - kw_n) // sw + 1 if sw == 2 else W
    Wp2, Hp2, g2, Lg2, Lc2 = _geom(Hout, Wout)
    Doutp = Dout + 2

    if sh == 1 and sw == 1:
        parts = {(0, 0): x}
        mparts = {(0, 0): m}
    else:
        x2 = x[:, :, g:g + Lg].reshape(Dp, Ci, Hp, Wp)
        m2 = m[:, 0, g:g + Lg].reshape(Dp, Hp, Wp)
        parts, mparts = {}, {}
        for hp_i in range(2 if sh == 2 else 1):
            xh = x2[:, :, hp_i::2, :] if sh == 2 else x2
            mh = m2[:, hp_i::2, :] if sh == 2 else m2
            for wp_i in range(2 if sw == 2 else 1):
                xs = xh[:, :, :, wp_i::2] if sw == 2 else xh
                ms = mh[:, :, wp_i::2] if sw == 2 else mh
                R, Cw = xs.shape[2], xs.shape[3]
                xs = jnp.pad(xs, ((0, 0), (0, 0), (0, 0), (0, Wp2 - Cw)))
                ms = jnp.pad(ms, ((0, 0), (0, 0), (0, Wp2 - Cw)))
                Lbody = R * Wp2
                Lpar = ((g2 + Lbody + g2 + Wp2 + 127) // 128) * 128
                xs = jnp.pad(xs.reshape(Dp, Ci, Lbody),
                             ((0, 0), (0, 0), (g2, Lpar - g2 - Lbody)))
                ms = jnp.pad(ms.reshape(Dp, Lbody),
                             ((0, 0), (g2, Lpar - g2 - Lbody)))
                parts[(hp_i, wp_i)] = xs
                mparts[(hp_i, wp_i)] = ms.reshape(Dp, 1, Lpar)

    block_key_to_idx, block_list, taps, w_list = {}, [], [], []
    for kd in range(kd_n):
        for kh in range(kh_n):
            for kw in range(kw_n):
                hp_i, ho = (kh % 2, kh // 2) if sh == 2 else (0, kh - ph + 1)
                wp_i, wo = (kw % 2, kw // 2) if sw == 2 else (0, kw - pw + 1)
                key = (hp_i, wp_i, kd)
                if key not in block_key_to_idx:
                    block_key_to_idx[key] = len(block_list)
                    block_list.append(key)
                taps.append((block_key_to_idx[key], ho * Wp2 + wo))
                w_list.append(Wf[:, :, kd, kh, kw])
    nb = len(block_list)
    w2 = jnp.concatenate(w_list, axis=0).astype(jnp.bfloat16)
    b3 = b.reshape(Co, 1).astype(jnp.float32)

    rr = jnp.arange(Hp2)
    cc = jnp.arange(Wp2)
    interior = (((rr >= 1) & (rr <= Hout))[:, None]
                & ((cc >= 1) & (cc <= Wout))[None, :]).astype(jnp.float32)
    interior = interior.reshape(1, Lg2)

    def _pmap(kd, d):
        return jnp.clip(sd * d + kd + 1 - pd - sd, 0, Dp - 1)

    specs = [
        pl.BlockSpec((len(taps) * Co, Ci), lambda d: (0, 0)),
        pl.BlockSpec((Co, 1), lambda d: (0, 0)),
        pl.BlockSpec((1, Lg2), lambda d: (0, 0)),
    ]
    args = [w2, b3, interior]
    for (hp_i, wp_i, kd) in block_list:
        a = mparts[(hp_i, wp_i)]
        specs.append(pl.BlockSpec(
            (1, 1, a.shape[2]),
            functools.partial(lambda kd, d: (_pmap(kd, d), 0, 0), kd)))
        args.append(a)
    for (hp_i, wp_i, kd) in block_list:
        a = parts[(hp_i, wp_i)]
        specs.append(pl.BlockSpec(
            (1, Ci, a.shape[2]),
            functools.partial(lambda kd, d: (_pmap(kd, d), 0, 0), kd)))
        args.append(a)

    out, mout = pl.pallas_call(
        functools.partial(_down_kernel, Co, taps, nb, g2, Lg2, Doutp),
        grid=(Doutp,),
        in_specs=specs,
        out_specs=[
            pl.BlockSpec((1, Co, Lc2), lambda d: (d, 0, 0)),
            pl.BlockSpec((1, 1, Lc2), lambda d: (d, 0, 0)),
        ],
        out_shape=[
            jax.ShapeDtypeStruct((Doutp, Co, Lc2), out_dtype),
            jax.ShapeDtypeStruct((Doutp, 1, Lc2), jnp.float32),
        ],
        compiler_params=pltpu.CompilerParams(
            dimension_semantics=("parallel",),
            vmem_limit_bytes=_VMEM),
    )(*args)
    return out, mout, (Hout, Wout)


def kernel(voxel_features, voxel_lin_idx, params):
    D, H, W = GD, GH, GW
    Wp, Hp, g, Lg, Lc = _geom(H, W)
    Dp = D + 2
    lin = voxel_lin_idx
    dd = lin // (H * W)
    rem = lin % (H * W)
    hh = rem // W
    ww = rem % W
    pos = (dd + 1) * Lc + g + (hh + 1) * Wp + (ww + 1)
    x = jnp.zeros((C_IN, Dp * Lc), jnp.float32).at[:, pos].set(
        voxel_features.T).astype(jnp.bfloat16).reshape(
            C_IN, Dp, Lc).transpose(1, 0, 2)
    m = jnp.zeros((Dp * Lc,), jnp.float32).at[pos].set(1.0).reshape(Dp, 1, Lc)

    fold = [_fold_bn(p) for p in params]

    x = _subm_layer(x, m, *fold[0], H, W)
    x = _subm_layer(x, m, *fold[1], H, W)
    x, m, (H, W) = _down_layer(x, m, *fold[2], (2, 2, 2), (1, 1, 1),
                               (3, 3, 3), H, W)
    x = _subm_layer(x, m, *fold[3], H, W)
    x = _subm_layer(x, m, *fold[4], H, W)
    x, m, (H, W) = _down_layer(x, m, *fold[5], (2, 2, 2), (1, 1, 1),
                               (3, 3, 3), H, W)
    x = _subm_layer(x, m, *fold[6], H, W)
    x = _subm_layer(x, m, *fold[7], H, W)
    x, m, (H, W) = _down_layer(x, m, *fold[8], (2, 2, 2), (0, 1, 1),
                               (3, 3, 3), H, W)
    x = _subm_layer(x, m, *fold[9], H, W)
    x = _subm_layer(x, m, *fold[10], H, W)
    x, m, (H, W) = _down_layer(x, m, *fold[11], (2, 1, 1), (0, 0, 0),
                               (3, 1, 1), H, W, out_dtype=jnp.float32)
    Wp, Hp, g, Lg, Lc = _geom(H, W)
    Dout = x.shape[0] - 2
    y = x[1:1 + Dout, :, g:g + Lg].reshape(Dout, x.shape[1], Hp, Wp)
    y = y[:, :, 1:1 + H, 1:1 + W].transpose(1, 0, 2, 3)
    return y[None]


# speedup vs baseline: 1.4265x; 1.0070x over previous
import functools

import jax
import jax.numpy as jnp
from jax.experimental import pallas as pl
from jax.experimental.pallas import tpu as pltpu

GD, GH, GW = 41, 320, 320
C_IN = 4
BN_EPS = 1e-3

_VMEM = 56 * 1024 * 1024


def _fold_bn(p):
    Wt, gamma, beta, mean, var = p
    scale = gamma * jax.lax.rsqrt(var + BN_EPS)
    Wf = Wt * scale[:, None, None, None, None]
    b = beta - mean * scale
    return Wf, b


def _geom(H, W):
    Wp, Hp = W + 2, H + 2
    g = Wp + 1
    Lg = Hp * Wp
    Lc = ((g + Lg + g + 127) // 128) * 128
    return Wp, Hp, g, Lg, Lc


def _subm_kernel(Co, Wp, g, Lg, Dp, w_ref, b_ref, m_ref, x0, x1, x2, o_ref):
    d = pl.program_id(0)

    CH = 8192

    @pl.when((d >= 1) & (d <= Dp - 2))
    def _compute():
        for c0 in range(0, Lg, CH):
            ch = min(CH, Lg - c0)
            acc = None
            t = 0
            for xr in (x0, x1, x2):
                for kh in range(3):
                    for kw in range(3):
                        base = kh * Wp + kw + c0
                        Wk = w_ref[t * Co:(t + 1) * Co, :]
                        xs = xr[0, :, base:base + ch]
                        y = jnp.dot(Wk, xs, preferred_element_type=jnp.float32)
                        acc = y if acc is None else acc + y
                        t += 1
            y = jnp.maximum(acc + b_ref[...], 0.0) * m_ref[0][:, g + c0:g + c0 + ch]
            o_ref[0, :, g + c0:g + c0 + ch] = y.astype(o_ref.dtype)
        o_ref[0, :, :g] = jnp.zeros((Co, g), o_ref.dtype)
        o_ref[0, :, g + Lg:] = jnp.zeros((Co, o_ref.shape[2] - g - Lg), o_ref.dtype)

    @pl.when((d < 1) | (d > Dp - 2))
    def _zero():
        o_ref[...] = jnp.zeros(o_ref.shape, o_ref.dtype)


def _subm_layer(x, m, Wf, b, H, W, out_dtype=jnp.bfloat16):
    Wp, Hp, g, Lg, Lc = _geom(H, W)
    Dp, Ci = x.shape[0], x.shape[1]
    Co = Wf.shape[0]
    w2 = jnp.transpose(Wf, (2, 3, 4, 0, 1)).reshape(27 * Co, Ci).astype(jnp.bfloat16)
    b3 = b.reshape(Co, 1).astype(jnp.float32)
    in_specs = [
        pl.BlockSpec((27 * Co, Ci), lambda d: (0, 0)),
        pl.BlockSpec((Co, 1), lambda d: (0, 0)),
        pl.BlockSpec((1, 1, Lc), lambda d: (d, 0, 0)),
    ]
    for kd in range(3):
        in_specs.append(pl.BlockSpec(
            (1, Ci, Lc),
            functools.partial(
                lambda kd, d: (jnp.clip(d + kd - 1, 0, Dp - 1), 0, 0), kd)))
    return pl.pallas_call(
        functools.partial(_subm_kernel, Co, Wp, g, Lg, Dp),
        grid=(Dp,),
        in_specs=in_specs,
        out_specs=pl.BlockSpec((1, Co, Lc), lambda d: (d, 0, 0)),
        out_shape=jax.ShapeDtypeStruct((Dp, Co, Lc), out_dtype),
        compiler_params=pltpu.CompilerParams(
            dimension_semantics=("parallel",),
            vmem_limit_bytes=_VMEM),
    )(w2, b3, m, x, x, x)


def _down_kernel(Co, taps, nb, g2, Lg2, Doutp, w_ref, b_ref, i_ref, *refs):
    m_refs = refs[:nb]
    x_refs = refs[nb:2 * nb]
    o_ref, mo_ref = refs[2 * nb], refs[2 * nb + 1]
    d = pl.program_id(0)

    CH = 8192

    @pl.when((d >= 1) & (d <= Doutp - 2))
    def _compute():
        Lc2 = o_ref.shape[2]
        for c0 in range(0, Lg2, CH):
            ch = min(CH, Lg2 - c0)
            acc = None
            mo = None
            for t, (bi, base) in enumerate(taps):
                Wk = w_ref[t * Co:(t + 1) * Co, :]
                xs = x_refs[bi][0, :, base + c0:base + c0 + ch]
                y = jnp.dot(Wk, xs, preferred_element_type=jnp.float32)
                acc = y if acc is None else acc + y
                mt = m_refs[bi][0][:, base + c0:base + c0 + ch]
                mo = mt if mo is None else jnp.maximum(mo, mt)
            mo = mo * i_ref[:, c0:c0 + ch]
            y = jnp.maximum(acc + b_ref[...], 0.0) * mo
            o_ref[0, :, g2 + c0:g2 + c0 + ch] = y.astype(o_ref.dtype)
            mo_ref[0, :, g2 + c0:g2 + c0 + ch] = mo
        o_ref[0, :, :g2] = jnp.zeros((Co, g2), o_ref.dtype)
        o_ref[0, :, g2 + Lg2:] = jnp.zeros((Co, Lc2 - g2 - Lg2), o_ref.dtype)
        mo_ref[0, :, :g2] = jnp.zeros((1, g2), mo_ref.dtype)
        mo_ref[0, :, g2 + Lg2:] = jnp.zeros((1, mo_ref.shape[2] - g2 - Lg2),
                                            mo_ref.dtype)

    @pl.when((d < 1) | (d > Doutp - 2))
    def _zero():
        o_ref[...] = jnp.zeros(o_ref.shape, o_ref.dtype)
        mo_ref[...] = jnp.zeros(mo_ref.shape, mo_ref.dtype)


def _down_layer(x, m, Wf, b, stride, pad, ksize, H, W,
                out_dtype=jnp.bfloat16):
    Wp, Hp, g, Lg, Lc = _geom(H, W)
    Dp, Ci = x.shape[0], x.shape[1]
    D = Dp - 2
    Co = Wf.shape[0]
    kd_n, kh_n, kw_n = ksize
    sd, sh, sw = stride
    pd, ph, pw = pad
    Dout = (D + 2 * pd - kd_n) // sd + 1
    Hout = (H + 2 * ph - kh_n) // sh + 1 if sh == 2 else H
    Wout = (W + 2 * pw - kw_n) // sw + 1 if sw == 2 else W
    Wp2, Hp2, g2, Lg2, Lc2 = _geom(Hout, Wout)
    Doutp = Dout + 2

    if sh == 1 and sw == 1:
        parts = {(0, 0): x}
        mparts = {(0, 0): m}
    else:
        x2 = x[:, :, g:g + Lg].reshape(Dp, Ci, Hp, Wp)
        m2 = m[:, 0, g:g + Lg].reshape(Dp, Hp, Wp)
        parts, mparts = {}, {}
        for hp_i in range(2 if sh == 2 else 1):
            xh = x2[:, :, hp_i::2, :] if sh == 2 else x2
            mh = m2[:, hp_i::2, :] if sh == 2 else m2
            for wp_i in range(2 if sw == 2 else 1):
                xs = xh[:, :, :, wp_i::2] if sw == 2 else xh
                ms = mh[:, :, wp_i::2] if sw == 2 else mh
                R, Cw = xs.shape[2], xs.shape[3]
                xs = jnp.pad(xs, ((0, 0), (0, 0), (0, 0), (0, Wp2 - Cw)))
                ms = jnp.pad(ms, ((0, 0), (0, 0), (0, Wp2 - Cw)))
                Lbody = R * Wp2
                Lpar = ((g2 + Lbody + g2 + Wp2 + 127) // 128) * 128
                xs = jnp.pad(xs.reshape(Dp, Ci, Lbody),
                             ((0, 0), (0, 0), (g2, Lpar - g2 - Lbody)))
                ms = jnp.pad(ms.reshape(Dp, Lbody),
                             ((0, 0), (g2, Lpar - g2 - Lbody)))
                parts[(hp_i, wp_i)] = xs
                mparts[(hp_i, wp_i)] = ms.reshape(Dp, 1, Lpar)

    block_key_to_idx, block_list, taps, w_list = {}, [], [], []
    for kd in range(kd_n):
        for kh in range(kh_n):
            for kw in range(kw_n):
                hp_i, ho = (kh % 2, kh // 2) if sh == 2 else (0, kh - ph + 1)
                wp_i, wo = (kw % 2, kw // 2) if sw == 2 else (0, kw - pw + 1)
                key = (hp_i, wp_i, kd)
                if key not in block_key_to_idx:
                    block_key_to_idx[key] = len(block_list)
                    block_list.append(key)
                taps.append((block_key_to_idx[key], ho * Wp2 + wo))
                w_list.append(Wf[:, :, kd, kh, kw])
    nb = len(block_list)
    w2 = jnp.concatenate(w_list, axis=0).astype(jnp.bfloat16)
    b3 = b.reshape(Co, 1).astype(jnp.float32)

    rr = jnp.arange(Hp2)
    cc = jnp.arange(Wp2)
    interior = (((rr >= 1) & (rr <= Hout))[:, None]
                & ((cc >= 1) & (cc <= Wout))[None, :]).astype(jnp.float32)
    interior = interior.reshape(1, Lg2)

    def _pmap(kd, d):
        return jnp.clip(sd * d + kd + 1 - pd - sd, 0, Dp - 1)

    specs = [
        pl.BlockSpec((len(taps) * Co, Ci), lambda d: (0, 0)),
        pl.BlockSpec((Co, 1), lambda d: (0, 0)),
        pl.BlockSpec((1, Lg2), lambda d: (0, 0)),
    ]
    args = [w2, b3, interior]
    for (hp_i, wp_i, kd) in block_list:
        a = mparts[(hp_i, wp_i)]
        specs.append(pl.BlockSpec(
            (1, 1, a.shape[2]),
            functools.partial(lambda kd, d: (_pmap(kd, d), 0, 0), kd)))
        args.append(a)
    for (hp_i, wp_i, kd) in block_list:
        a = parts[(hp_i, wp_i)]
        specs.append(pl.BlockSpec(
            (1, Ci, a.shape[2]),
            functools.partial(lambda kd, d: (_pmap(kd, d), 0, 0), kd)))
        args.append(a)

    out, mout = pl.pallas_call(
        functools.partial(_down_kernel, Co, taps, nb, g2, Lg2, Doutp),
        grid=(Doutp,),
        in_specs=specs,
        out_specs=[
            pl.BlockSpec((1, Co, Lc2), lambda d: (d, 0, 0)),
            pl.BlockSpec((1, 1, Lc2), lambda d: (d, 0, 0)),
        ],
        out_shape=[
            jax.ShapeDtypeStruct((Doutp, Co, Lc2), out_dtype),
            jax.ShapeDtypeStruct((Doutp, 1, Lc2), jnp.float32),
        ],
        compiler_params=pltpu.CompilerParams(
            dimension_semantics=("parallel",),
            vmem_limit_bytes=_VMEM),
    )(*args)
    return out, mout, (Hout, Wout)


def kernel(voxel_features, voxel_lin_idx, params):
    D, H, W = GD, GH, GW
    Wp, Hp, g, Lg, Lc = _geom(H, W)
    Dp = D + 2
    lin = voxel_lin_idx
    dd = lin // (H * W)
    rem = lin % (H * W)
    hh = rem // W
    ww = rem % W
    pos = (dd + 1) * Lc + g + (hh + 1) * Wp + (ww + 1)
    x = jnp.zeros((C_IN, Dp * Lc), jnp.float32).at[:, pos].set(
        voxel_features.T).astype(jnp.bfloat16).reshape(
            C_IN, Dp, Lc).transpose(1, 0, 2)
    m = jnp.zeros((Dp * Lc,), jnp.float32).at[pos].set(1.0).reshape(Dp, 1, Lc)

    fold = [_fold_bn(p) for p in params]

    x = _subm_layer(x, m, *fold[0], H, W)
    x = _subm_layer(x, m, *fold[1], H, W)
    x, m, (H, W) = _down_layer(x, m, *fold[2], (2, 2, 2), (1, 1, 1),
                               (3, 3, 3), H, W)
    x = _subm_layer(x, m, *fold[3], H, W)
    x = _subm_layer(x, m, *fold[4], H, W)
    x, m, (H, W) = _down_layer(x, m, *fold[5], (2, 2, 2), (1, 1, 1),
                               (3, 3, 3), H, W)
    x = _subm_layer(x, m, *fold[6], H, W)
    x = _subm_layer(x, m, *fold[7], H, W)
    x, m, (H, W) = _down_layer(x, m, *fold[8], (2, 2, 2), (0, 1, 1),
                               (3, 3, 3), H, W)
    x = _subm_layer(x, m, *fold[9], H, W)
    x = _subm_layer(x, m, *fold[10], H, W)
    x, m, (H, W) = _down_layer(x, m, *fold[11], (2, 1, 1), (0, 0, 0),
                               (3, 1, 1), H, W, out_dtype=jnp.float32)
    Wp, Hp, g, Lg, Lc = _geom(H, W)
    Dout = x.shape[0] - 2
    y = x[1:1 + Dout, :, g:g + Lg].reshape(Dout, x.shape[1], Hp, Wp)
    y = y[:, :, 1:1 + H, 1:1 + W].transpose(1, 0, 2, 3)
    return y[None]
